# baseline (speedup 1.0000x reference)
import functools

import jax
import jax.numpy as jnp
from jax import lax
from jax.experimental import pallas as pl
from jax.experimental.pallas import tpu as pltpu
from jax.experimental.pallas import tpu_sc as plsc

N = 10000
E = 320000
D_IN = 128
D_HID = 70
DP = 128
NZV = 5
N_CLS = 4
OP = 128

NC = 2
NS = 16
NW = NC * NS
EPW = E // NW
CH = 80
NCHUNK = EPW // CH
B = 6
NBODY = (NCHUNK - 5) // (2 * B)
TAIL0 = NBODY * 2 * B
NPAD = 10240
RPT = NPAD // NS

_mesh = plsc.VectorSubcoreMesh(core_axis_name="c", subcore_axis_name="s")


@functools.partial(
    pl.kernel,
    out_type=jax.ShapeDtypeStruct((NC, NPAD, DP), jnp.float32),
    mesh=_mesh,
    scratch_types=[
        pltpu.VMEM((B, 3, CH), jnp.int32),
        pltpu.VMEM((B, 3, CH), jnp.int32),
        pltpu.VMEM((CH, DP), jnp.float32),
        pltpu.VMEM((CH, DP), jnp.float32),
        pltpu.VMEM((CH, DP), jnp.float32),
        pltpu.VMEM((CH, DP), jnp.float32),
        pltpu.VMEM_SHARED((NPAD, DP), jnp.float32),
        pltpu.SemaphoreType.DMA,
        pltpu.SemaphoreType.DMA,
        [pltpu.SemaphoreType.DMA] * 4,
        [pltpu.SemaphoreType.DMA] * 4,
    ],
)
def _spmm(y_hbm, ed_hbm, out_hbm,
          ebuf0, ebuf1, msg0, msg1, msg2, msg3, acc_sh, se0, se1, sg, ss):
    cid = lax.axis_index("c")
    sid = lax.axis_index("s")
    wid = sid * NC + cid

    zeros = jnp.zeros((16,), jnp.float32)

    def zfill(i, carry):
        for j in range(DP // 16):
            msg0[i, pl.ds(j * 16, 16)] = zeros
        return carry

    lax.fori_loop(0, CH, zfill, 0)

    row0 = sid * RPT
    for k in range(RPT // CH):
        pltpu.sync_copy(msg0, acc_sh.at[pl.ds(row0 + k * CH, CH)])

    plsc.subcore_barrier()

    msgs = (msg0, msg1, msg2, msg3)

    def ed_issue(blk0, nchunks, ebuf, sem):
        pltpu.async_copy(ed_hbm.at[wid, pl.ds(blk0, nchunks)],
                         ebuf.at[pl.ds(0, nchunks)], sem)

    def ed_wait(nchunks, ebuf, sem):
        pltpu.make_async_copy(ed_hbm.at[wid, pl.ds(0, nchunks)],
                              ebuf.at[pl.ds(0, nchunks)], sem).wait()

    def g_issue(ebuf, k, msg, sem):
        pltpu.async_copy(y_hbm.at[ebuf.at[k, 0]], msg, sem)

    def g_wait(ebuf, k, msg, sem):
        pltpu.make_async_copy(y_hbm.at[ebuf.at[k, 0]], msg, sem).wait()

    def scale(ebuf, k, msg):
        @plsc.parallel_loop(0, CH // 16, step=1)
        def scale_g(g):
            ewg = lax.bitcast_convert_type(
                ebuf[k, 2, pl.ds(g * 16, 16)], jnp.float32)
            for lane in range(16):
                sconst = ewg[lane]
                e = g * 16 + lane
                for j in range(NZV):
                    sl = pl.ds(j * 16, 16)
                    msg[e, sl] = msg[e, sl] * sconst

    def sc_issue(ebuf, k, m):
        pltpu.async_copy(msgs[m], acc_sh.at[ebuf.at[k, 1]], ss[m], add=True)

    def sc_wait(m):
        pltpu.make_async_copy(msgs[m], acc_sh.at[ebuf0.at[0, 1]],
                              ss[m]).wait()

    def half(ebuf, nxt_ebuf, nxt_k, phase, guard_u=None):
        for k in range(B):
            m = (k + phase) % 4
            m1 = (k + 1 + phase) % 4
            if k >= 3:
                sc_wait(m1)
            if k < B - 1:
                g_issue(ebuf, k + 1, msgs[m1], sg[m1])
            else:
                if guard_u is None:
                    g_issue(nxt_ebuf, nxt_k, msgs[m1], sg[m1])
                else:
                    guard_u()
                    g_issue(nxt_ebuf, nxt_k, msgs[m1], sg[m1])
            g_wait(ebuf, k, msgs[m], sg[m])
            scale(ebuf, k, msgs[m])
            sc_issue(ebuf, k, m)
        for k in range(B - 3, B):
            sc_wait((k + phase) % 4)

    ed_issue(0, B, ebuf0, se0)
    ed_wait(B, ebuf0, se0)
    g_issue(ebuf0, 0, msg0, sg[0])
    ed_issue(B, B, ebuf1, se1)

    def body(u, carry):
        c0 = u * 2 * B

        def w1():
            ed_wait(B, ebuf1, se1)

        half(ebuf0, ebuf1, 0, 0, guard_u=w1)
        @pl.when(u < NBODY - 1)
        def _():
            ed_issue(c0 + 2 * B, B, ebuf0, se0)

        @pl.when(u == NBODY - 1)
        def _():
            ed_issue(TAIL0, 5, ebuf0, se0)

        def w0():
            @pl.when(u < NBODY - 1)
            def _():
                ed_wait(B, ebuf0, se0)

            @pl.when(u == NBODY - 1)
            def _():
                ed_wait(5, ebuf0, se0)

        half(ebuf1, ebuf0, 0, 2, guard_u=w0)
        @pl.when(u < NBODY - 1)
        def _():
            ed_issue(c0 + 3 * B, B, ebuf1, se1)

        return carry

    lax.fori_loop(0, NBODY, body, 0)

    NT = NCHUNK - TAIL0
    for k in range(NT):
        m = k % 4
        m1 = (k + 1) % 4
        if k >= 3:
            sc_wait(m1)
        if k < NT - 1:
            g_issue(ebuf0, k + 1, msgs[m1], sg[m1])
        g_wait(ebuf0, k, msgs[m], sg[m])
        scale(ebuf0, k, msgs[m])
        sc_issue(ebuf0, k, m)
    for k in range(NT - 3, NT):
        sc_wait(k % 4)

    plsc.subcore_barrier()
    pltpu.sync_copy(acc_sh.at[pl.ds(row0, RPT)],
                    out_hbm.at[cid, pl.ds(row0, RPT)])


def _mm_body(x_ref, w_ref, o_ref):
    o_ref[...] = jnp.dot(x_ref[...], w_ref[...],
                         preferred_element_type=jnp.float32)


def _fuse_body(y_ref, a0_ref, a1_ref, b_ref, w_ref, o_ref):
    h = jnp.maximum(y_ref[...] + a0_ref[...] + a1_ref[...] + b_ref[...], 0.0)
    o_ref[...] = jnp.dot(h, w_ref[...], preferred_element_type=jnp.float32)


def _final_body(y_ref, a0_ref, a1_ref, b_ref, w_ref, b4_ref, o_ref):
    h = jnp.maximum(y_ref[...] + a0_ref[...] + a1_ref[...] + b_ref[...], 0.0)
    o_ref[...] = (jnp.dot(h, w_ref[...], preferred_element_type=jnp.float32)
                  + b4_ref[...])


_mm = pl.pallas_call(
    _mm_body, out_shape=jax.ShapeDtypeStruct((N, DP), jnp.float32))
_fuse = pl.pallas_call(
    _fuse_body, out_shape=jax.ShapeDtypeStruct((N, DP), jnp.float32))
_final = pl.pallas_call(
    _final_body, out_shape=jax.ShapeDtypeStruct((N, OP), jnp.float32))


def _pad_w(W, rows, cols):
    return jnp.zeros((rows, cols), jnp.float32).at[:W.shape[0], :W.shape[1]].set(W)


def _pad_b(b, cols):
    return jnp.zeros((1, cols), jnp.float32).at[0, :b.shape[0]].set(b)


def kernel(x, edge_index, edge_weights, W1, b1, W2, b2, W3, b3, W4, b4):
    src = edge_index[0].astype(jnp.int32).reshape(NW, NCHUNK, CH)
    dst = edge_index[1].astype(jnp.int32).reshape(NW, NCHUNK, CH)
    ewb = jax.lax.bitcast_convert_type(
        edge_weights.astype(jnp.float32), jnp.int32).reshape(NW, NCHUNK, CH)
    ed = jnp.stack([src, dst, ewb], axis=2)

    W1p = _pad_w(W1, D_IN, DP)
    W2p = _pad_w(W2, DP, DP)
    W3p = _pad_w(W3, DP, DP)
    W4p = _pad_w(W4, DP, OP)
    b1p = _pad_b(b1, DP)
    b2p = _pad_b(b2, DP)
    b3p = _pad_b(b3, DP)
    b4p = _pad_b(b4, OP)

    y1 = _mm(x.astype(jnp.float32), W1p)
    a1 = _spmm(y1, ed)
    y2 = _fuse(y1, a1[0, :N], a1[1, :N], b1p, W2p)
    a2 = _spmm(y2, ed)
    y3 = _fuse(y2, a2[0, :N], a2[1, :N], b2p, W3p)
    a3 = _spmm(y3, ed)
    out = _final(y3, a3[0, :N], a3[1, :N], b3p, W4p, b4p)
    return out[:, :N_CLS]


# speedup vs baseline: 1.1869x; 1.1869x over previous
import functools

import jax
import jax.numpy as jnp
from jax import lax
from jax.experimental import pallas as pl
from jax.experimental.pallas import tpu as pltpu
from jax.experimental.pallas import tpu_sc as plsc

N = 10000
E = 320000
D_IN = 128
D_HID = 70
DP = 128
NZV = 5
N_CLS = 4
OP = 128

NC = 2
NS = 16
NW = NC * NS
EPW = E // NW
CH = 80
NCHUNK = EPW // CH
B = 6
NBODY = (NCHUNK - 5) // (2 * B)
TAIL0 = NBODY * 2 * B
NPAD = 10240
RPT = NPAD // NS

_mesh = plsc.VectorSubcoreMesh(core_axis_name="c", subcore_axis_name="s")


@functools.partial(
    pl.kernel,
    out_type=jax.ShapeDtypeStruct((NC, NPAD, DP), jnp.float32),
    mesh=_mesh,
    scratch_types=[
        pltpu.VMEM((B, 3, CH), jnp.int32),
        pltpu.VMEM((B, 3, CH), jnp.int32),
        pltpu.VMEM((CH, DP), jnp.float32),
        pltpu.VMEM((CH, DP), jnp.float32),
        pltpu.VMEM((CH, DP), jnp.float32),
        pltpu.VMEM((CH, DP), jnp.float32),
        pltpu.VMEM_SHARED((NPAD, DP), jnp.float32),
        pltpu.SemaphoreType.DMA,
        pltpu.SemaphoreType.DMA,
        [pltpu.SemaphoreType.DMA] * 4,
        [pltpu.SemaphoreType.DMA] * 4,
    ],
)
def _spmm(y_hbm, ed_hbm, out_hbm,
          ebuf0, ebuf1, msg0, msg1, msg2, msg3, acc_sh, se0, se1, sg, ss):
    cid = lax.axis_index("c")
    sid = lax.axis_index("s")
    wid = sid * NC + cid

    zeros = jnp.zeros((16,), jnp.float32)

    def zfill(i, carry):
        for j in range(DP // 16):
            msg0[i, pl.ds(j * 16, 16)] = zeros
        return carry

    lax.fori_loop(0, CH, zfill, 0)

    row0 = sid * RPT
    for k in range(RPT // CH):
        pltpu.sync_copy(msg0, acc_sh.at[pl.ds(row0 + k * CH, CH)])

    plsc.subcore_barrier()

    msgs = (msg0, msg1, msg2, msg3)

    def ed_issue(blk0, nchunks, ebuf, sem):
        pltpu.async_copy(ed_hbm.at[wid, pl.ds(blk0, nchunks)],
                         ebuf.at[pl.ds(0, nchunks)], sem)

    def ed_wait(nchunks, ebuf, sem):
        pltpu.make_async_copy(ed_hbm.at[wid, pl.ds(0, nchunks)],
                              ebuf.at[pl.ds(0, nchunks)], sem).wait()

    def g_issue(ebuf, k, msg, sem):
        pltpu.async_copy(y_hbm.at[ebuf.at[k, 0]], msg, sem)

    def g_wait(ebuf, k, msg, sem):
        pltpu.make_async_copy(y_hbm.at[ebuf.at[k, 0]], msg, sem).wait()

    def scale(ebuf, k, msg):
        def scale_g(g, c2):
            ewg = lax.bitcast_convert_type(
                ebuf[k, 2, pl.ds(g * 16, 16)], jnp.float32)
            for lane in range(16):
                sconst = ewg[lane]
                e = g * 16 + lane
                for j in range(NZV):
                    sl = pl.ds(j * 16, 16)
                    msg[e, sl] = msg[e, sl] * sconst
            return c2

        lax.fori_loop(0, CH // 16, scale_g, 0)

    def sc_issue(ebuf, k, m):
        pltpu.async_copy(msgs[m], acc_sh.at[ebuf.at[k, 1]], ss[m], add=True)

    def sc_wait(m):
        pltpu.make_async_copy(msgs[m], acc_sh.at[ebuf0.at[0, 1]],
                              ss[m]).wait()

    def half(ebuf, nxt_ebuf, phase, ed_wait_nxt, first=None, post_k1=None):
        for k in range(B):
            m = (k + phase) % 4
            m2 = (k + 2 + phase) % 4
            if first is not None and k < 2:
                @pl.when(first > 0)
                def _():
                    sc_wait(m2)
            else:
                sc_wait(m2)
            if k == 1 and post_k1 is not None:
                post_k1()
            if k < B - 2:
                g_issue(ebuf, k + 2, msgs[m2], sg[m2])
            elif k == B - 2:
                ed_wait_nxt()
                g_issue(nxt_ebuf, 0, msgs[m2], sg[m2])
            else:
                g_issue(nxt_ebuf, 1, msgs[m2], sg[m2])
            g_wait(ebuf, k, msgs[m], sg[m])
            scale(ebuf, k, msgs[m])
            sc_issue(ebuf, k, m)

    ed_issue(0, B, ebuf0, se0)
    ed_wait(B, ebuf0, se0)
    g_issue(ebuf0, 0, msg0, sg[0])
    g_issue(ebuf0, 1, msg1, sg[1])
    ed_issue(B, B, ebuf1, se1)

    def body(u, carry):
        c0 = u * 2 * B

        half(ebuf0, ebuf1, 0,
             lambda: ed_wait(B, ebuf1, se1), first=u)

        def post_k1():
            @pl.when(u < NBODY - 1)
            def _():
                ed_issue(c0 + 2 * B, B, ebuf0, se0)

            @pl.when(u == NBODY - 1)
            def _():
                ed_issue(TAIL0, 5, ebuf0, se0)

        def ed_wait0():
            @pl.when(u < NBODY - 1)
            def _():
                ed_wait(B, ebuf0, se0)

            @pl.when(u == NBODY - 1)
            def _():
                ed_wait(5, ebuf0, se0)

        half(ebuf1, ebuf0, 2, ed_wait0, post_k1=post_k1)
        @pl.when(u < NBODY - 1)
        def _():
            ed_issue(c0 + 3 * B, B, ebuf1, se1)

        return carry

    lax.fori_loop(0, NBODY, body, 0)

    NT = NCHUNK - TAIL0
    for k in range(NT):
        m = k % 4
        m2 = (k + 2) % 4
        sc_wait(m2)
        if k + 2 < NT:
            g_issue(ebuf0, k + 2, msgs[m2], sg[m2])
        g_wait(ebuf0, k, msgs[m], sg[m])
        scale(ebuf0, k, msgs[m])
        sc_issue(ebuf0, k, m)
    for k in range(NT - 2, NT):
        sc_wait(k % 4)

    plsc.subcore_barrier()
    pltpu.sync_copy(acc_sh.at[pl.ds(row0, RPT)],
                    out_hbm.at[cid, pl.ds(row0, RPT)])


def _mm_body(x_ref, w_ref, o_ref):
    o_ref[...] = jnp.dot(x_ref[...], w_ref[...],
                         preferred_element_type=jnp.float32)


def _fuse_body(y_ref, a0_ref, a1_ref, b_ref, w_ref, o_ref):
    h = jnp.maximum(y_ref[...] + a0_ref[...] + a1_ref[...] + b_ref[...], 0.0)
    o_ref[...] = jnp.dot(h, w_ref[...], preferred_element_type=jnp.float32)


def _final_body(y_ref, a0_ref, a1_ref, b_ref, w_ref, b4_ref, o_ref):
    h = jnp.maximum(y_ref[...] + a0_ref[...] + a1_ref[...] + b_ref[...], 0.0)
    o_ref[...] = (jnp.dot(h, w_ref[...], preferred_element_type=jnp.float32)
                  + b4_ref[...])


_mm = pl.pallas_call(
    _mm_body, out_shape=jax.ShapeDtypeStruct((N, DP), jnp.float32))
_fuse = pl.pallas_call(
    _fuse_body, out_shape=jax.ShapeDtypeStruct((N, DP), jnp.float32))
_final = pl.pallas_call(
    _final_body, out_shape=jax.ShapeDtypeStruct((N, OP), jnp.float32))


def _pad_w(W, rows, cols):
    return jnp.zeros((rows, cols), jnp.float32).at[:W.shape[0], :W.shape[1]].set(W)


def _pad_b(b, cols):
    return jnp.zeros((1, cols), jnp.float32).at[0, :b.shape[0]].set(b)


def kernel(x, edge_index, edge_weights, W1, b1, W2, b2, W3, b3, W4, b4):
    src = edge_index[0].astype(jnp.int32).reshape(NW, NCHUNK, CH)
    dst = edge_index[1].astype(jnp.int32).reshape(NW, NCHUNK, CH)
    ewb = jax.lax.bitcast_convert_type(
        edge_weights.astype(jnp.float32), jnp.int32).reshape(NW, NCHUNK, CH)
    ed = jnp.stack([src, dst, ewb], axis=2)

    W1p = _pad_w(W1, D_IN, DP)
    W2p = _pad_w(W2, DP, DP)
    W3p = _pad_w(W3, DP, DP)
    W4p = _pad_w(W4, DP, OP)
    b1p = _pad_b(b1, DP)
    b2p = _pad_b(b2, DP)
    b3p = _pad_b(b3, DP)
    b4p = _pad_b(b4, OP)

    y1 = _mm(x.astype(jnp.float32), W1p)
    a1 = _spmm(y1, ed)
    y2 = _fuse(y1, a1[0, :N], a1[1, :N], b1p, W2p)
    a2 = _spmm(y2, ed)
    y3 = _fuse(y2, a2[0, :N], a2[1, :N], b2p, W3p)
    a3 = _spmm(y3, ed)
    out = _final(y3, a3[0, :N], a3[1, :N], b3p, W4p, b4p)
    return out[:, :N_CLS]


# speedup vs baseline: 1.1889x; 1.0017x over previous
import functools

import jax
import jax.numpy as jnp
from jax import lax
from jax.experimental import pallas as pl
from jax.experimental.pallas import tpu as pltpu
from jax.experimental.pallas import tpu_sc as plsc

N = 10000
E = 320000
D_IN = 128
D_HID = 70
DP = 128
NZV = 5
N_CLS = 4
OP = 128

NC = 2
NS = 16
NW = NC * NS
EPW = E // NW
CH = 80
NCHUNK = EPW // CH
B = 6
NBODY = (NCHUNK - 5) // (2 * B)
TAIL0 = NBODY * 2 * B
NPAD = 10240
RPT = NPAD // NS

_mesh = plsc.VectorSubcoreMesh(core_axis_name="c", subcore_axis_name="s")


@functools.partial(
    pl.kernel,
    out_type=jax.ShapeDtypeStruct((NC, NPAD, DP), jnp.float32),
    mesh=_mesh,
    scratch_types=[
        pltpu.VMEM((B, 3, CH), jnp.int32),
        pltpu.VMEM((B, 3, CH), jnp.int32),
        pltpu.VMEM((CH, DP), jnp.float32),
        pltpu.VMEM((CH, DP), jnp.float32),
        pltpu.VMEM((CH, DP), jnp.float32),
        pltpu.VMEM((CH, DP), jnp.float32),
        pltpu.VMEM_SHARED((NPAD, DP), jnp.float32),
        pltpu.SemaphoreType.DMA,
        pltpu.SemaphoreType.DMA,
        [pltpu.SemaphoreType.DMA] * 4,
        [pltpu.SemaphoreType.DMA] * 4,
    ],
)
def _spmm(y_hbm, ed_hbm, out_hbm,
          ebuf0, ebuf1, msg0, msg1, msg2, msg3, acc_sh, se0, se1, sg, ss):
    cid = lax.axis_index("c")
    sid = lax.axis_index("s")
    wid = sid * NC + cid

    pltpu.async_copy(ed_hbm.at[wid, pl.ds(0, B)], ebuf0.at[pl.ds(0, B)], se0)
    pltpu.async_copy(ed_hbm.at[wid, pl.ds(B, B)], ebuf1.at[pl.ds(0, B)], se1)

    zeros = jnp.zeros((16,), jnp.float32)

    def zfill(i, carry):
        for j in range(DP // 16):
            msg0[i, pl.ds(j * 16, 16)] = zeros
        return carry

    lax.fori_loop(0, CH, zfill, 0)

    row0 = sid * RPT
    for k in range(RPT // CH):
        pltpu.sync_copy(msg0, acc_sh.at[pl.ds(row0 + k * CH, CH)])

    plsc.subcore_barrier()

    msgs = (msg0, msg1, msg2, msg3)

    def ed_issue(blk0, nchunks, ebuf, sem):
        pltpu.async_copy(ed_hbm.at[wid, pl.ds(blk0, nchunks)],
                         ebuf.at[pl.ds(0, nchunks)], sem)

    def ed_wait(nchunks, ebuf, sem):
        pltpu.make_async_copy(ed_hbm.at[wid, pl.ds(0, nchunks)],
                              ebuf.at[pl.ds(0, nchunks)], sem).wait()

    def g_issue(ebuf, k, msg, sem):
        pltpu.async_copy(y_hbm.at[ebuf.at[k, 0]], msg, sem)

    def g_wait(ebuf, k, msg, sem):
        pltpu.make_async_copy(y_hbm.at[ebuf.at[k, 0]], msg, sem).wait()

    def scale(ebuf, k, msg):
        def scale_g(g, c2):
            ewg = lax.bitcast_convert_type(
                ebuf[k, 2, pl.ds(g * 16, 16)], jnp.float32)
            for lane in range(16):
                sconst = ewg[lane]
                e = g * 16 + lane
                for j in range(NZV):
                    sl = pl.ds(j * 16, 16)
                    msg[e, sl] = msg[e, sl] * sconst
            return c2

        lax.fori_loop(0, CH // 16, scale_g, 0)

    def sc_issue(ebuf, k, m):
        pltpu.async_copy(msgs[m], acc_sh.at[ebuf.at[k, 1]], ss[m], add=True)

    def sc_wait(m):
        pltpu.make_async_copy(msgs[m], acc_sh.at[ebuf0.at[0, 1]],
                              ss[m]).wait()

    def half(ebuf, nxt_ebuf, phase, ed_wait_nxt, first=None, post_k1=None):
        for k in range(B):
            m = (k + phase) % 4
            m2 = (k + 2 + phase) % 4
            if first is not None and k < 2:
                @pl.when(first > 0)
                def _():
                    sc_wait(m2)
            else:
                sc_wait(m2)
            if k == 1 and post_k1 is not None:
                post_k1()
            if k < B - 2:
                g_issue(ebuf, k + 2, msgs[m2], sg[m2])
            elif k == B - 2:
                ed_wait_nxt()
                g_issue(nxt_ebuf, 0, msgs[m2], sg[m2])
            else:
                g_issue(nxt_ebuf, 1, msgs[m2], sg[m2])
            g_wait(ebuf, k, msgs[m], sg[m])
            scale(ebuf, k, msgs[m])
            sc_issue(ebuf, k, m)

    ed_wait(B, ebuf0, se0)
    g_issue(ebuf0, 0, msg0, sg[0])
    g_issue(ebuf0, 1, msg1, sg[1])

    def body(u, carry):
        c0 = u * 2 * B

        half(ebuf0, ebuf1, 0,
             lambda: ed_wait(B, ebuf1, se1), first=u)

        def post_k1():
            @pl.when(u < NBODY - 1)
            def _():
                ed_issue(c0 + 2 * B, B, ebuf0, se0)

            @pl.when(u == NBODY - 1)
            def _():
                ed_issue(TAIL0, 5, ebuf0, se0)

        def ed_wait0():
            @pl.when(u < NBODY - 1)
            def _():
                ed_wait(B, ebuf0, se0)

            @pl.when(u == NBODY - 1)
            def _():
                ed_wait(5, ebuf0, se0)

        half(ebuf1, ebuf0, 2, ed_wait0, post_k1=post_k1)
        @pl.when(u < NBODY - 1)
        def _():
            ed_issue(c0 + 3 * B, B, ebuf1, se1)

        return carry

    lax.fori_loop(0, NBODY, body, 0)

    NT = NCHUNK - TAIL0
    for k in range(NT):
        m = k % 4
        m2 = (k + 2) % 4
        sc_wait(m2)
        if k + 2 < NT:
            g_issue(ebuf0, k + 2, msgs[m2], sg[m2])
        g_wait(ebuf0, k, msgs[m], sg[m])
        scale(ebuf0, k, msgs[m])
        sc_issue(ebuf0, k, m)
    for k in range(NT - 2, NT):
        sc_wait(k % 4)

    plsc.subcore_barrier()
    pltpu.sync_copy(acc_sh.at[pl.ds(row0, RPT)],
                    out_hbm.at[cid, pl.ds(row0, RPT)])


def _mm_body(x_ref, w_ref, o_ref):
    o_ref[...] = jnp.dot(x_ref[...], w_ref[...],
                         preferred_element_type=jnp.float32)


def _fuse_body(y_ref, a0_ref, a1_ref, b_ref, w_ref, o_ref):
    h = jnp.maximum(y_ref[...] + a0_ref[...] + a1_ref[...] + b_ref[...], 0.0)
    o_ref[...] = jnp.dot(h, w_ref[...], preferred_element_type=jnp.float32)


def _final_body(y_ref, a0_ref, a1_ref, b_ref, w_ref, b4_ref, o_ref):
    h = jnp.maximum(y_ref[...] + a0_ref[...] + a1_ref[...] + b_ref[...], 0.0)
    o_ref[...] = (jnp.dot(h, w_ref[...], preferred_element_type=jnp.float32)
                  + b4_ref[...])


_mm = pl.pallas_call(
    _mm_body, out_shape=jax.ShapeDtypeStruct((N, DP), jnp.float32))
_fuse = pl.pallas_call(
    _fuse_body, out_shape=jax.ShapeDtypeStruct((N, DP), jnp.float32))
_final = pl.pallas_call(
    _final_body, out_shape=jax.ShapeDtypeStruct((N, OP), jnp.float32))


def _pad_w(W, rows, cols):
    return jnp.zeros((rows, cols), jnp.float32).at[:W.shape[0], :W.shape[1]].set(W)


def _pad_b(b, cols):
    return jnp.zeros((1, cols), jnp.float32).at[0, :b.shape[0]].set(b)


def kernel(x, edge_index, edge_weights, W1, b1, W2, b2, W3, b3, W4, b4):
    src = edge_index[0].astype(jnp.int32).reshape(NW, NCHUNK, CH)
    dst = edge_index[1].astype(jnp.int32).reshape(NW, NCHUNK, CH)
    ewb = jax.lax.bitcast_convert_type(
        edge_weights.astype(jnp.float32), jnp.int32).reshape(NW, NCHUNK, CH)
    ed = jnp.stack([src, dst, ewb], axis=2)

    W1p = _pad_w(W1, D_IN, DP)
    W2p = _pad_w(W2, DP, DP)
    W3p = _pad_w(W3, DP, DP)
    W4p = _pad_w(W4, DP, OP)
    b1p = _pad_b(b1, DP)
    b2p = _pad_b(b2, DP)
    b3p = _pad_b(b3, DP)
    b4p = _pad_b(b4, OP)

    y1 = _mm(x.astype(jnp.float32), W1p)
    a1 = _spmm(y1, ed)
    y2 = _fuse(y1, a1[0, :N], a1[1, :N], b1p, W2p)
    a2 = _spmm(y2, ed)
    y3 = _fuse(y2, a2[0, :N], a2[1, :N], b2p, W3p)
    a3 = _spmm(y3, ed)
    out = _final(y3, a3[0, :N], a3[1, :N], b3p, W4p, b4p)
    return out[:, :N_CLS]
